# Initial kernel scaffold; baseline (speedup 1.0000x reference)
import jax
import jax.numpy as jnp
from jax.experimental import pallas as pl


def kernel(x, edge_index, W1, a_src1, a_dst1, b1, W2, a_src2, a_dst2, b2):
    raise NotImplementedError("write your pallas kernel here")


# speedup vs baseline: 8.3768x; 8.3768x over previous
import functools

import jax
import jax.numpy as jnp
from jax import lax
from jax.experimental import pallas as pl
from jax.experimental.pallas import tpu as pltpu
from jax.experimental.pallas import tpu_sc as plsc

_N = 10000
_NPAD = 10240
_E = 320000
_D = 128
_HD = 64
_NB = _HD // 16
_C = 64
_EPT = 20736
_NCH = _EPT // _C
_EPAD = 16 * _EPT
_SEG = _NPAD // 16
_BM = 80
_EPS = 1e-16


def _proj_body(x_ref, w_ref, h_ref, s_ref):
    t = jnp.dot(x_ref[...], w_ref[...], preferred_element_type=jnp.float32)
    h_ref[...] = t[:, :128]
    s_ref[...] = t[:, 128:]


def _proj(x, wext):
    return pl.pallas_call(
        _proj_body,
        grid=(_NPAD // _BM,),
        in_specs=[pl.BlockSpec((_BM, 128),
                               lambda i: (jnp.minimum(i, _N // _BM - 1), 0)),
                  pl.BlockSpec((128, 256), lambda i: (0, 0))],
        out_specs=[pl.BlockSpec((_BM, 128), lambda i: (i, 0)),
                   pl.BlockSpec((_BM, 128), lambda i: (i, 0))],
        out_shape=[jax.ShapeDtypeStruct((_NPAD, 128), jnp.float32),
                   jax.ShapeDtypeStruct((_NPAD, 128), jnp.float32)],
    )(x, wext)


def _halves(a_ref):
    xlo = jnp.concatenate([a_ref[0, j] for j in range(_NB)], axis=1)
    xhi = jnp.concatenate([a_ref[1, j] for j in range(_NB)], axis=1)
    return xlo, xhi


def _proj2_body(a_ref, d_ref, b_ref, w_ref, h_ref, s_ref):
    recip = 1.0 / (d_ref[0][:, 0:1] + _EPS)
    alo, ahi = _halves(a_ref)
    xlo = jnp.maximum(alo * recip + b_ref[:, :_HD], 0.0)
    xhi = jnp.maximum(ahi * recip + b_ref[:, _HD:], 0.0)
    t = (jnp.dot(xlo, w_ref[:_HD, :], preferred_element_type=jnp.float32)
         + jnp.dot(xhi, w_ref[_HD:, :], preferred_element_type=jnp.float32))
    h_ref[...] = t[:, :128]
    s_ref[...] = t[:, 128:]


def _proj2(accp, denp, brow, wext):
    return pl.pallas_call(
        _proj2_body,
        grid=(_NPAD // _BM,),
        in_specs=[pl.BlockSpec((2, _NB, _BM, 16), lambda i: (0, 0, i, 0)),
                  pl.BlockSpec((2, _BM, 16), lambda i: (0, i, 0)),
                  pl.BlockSpec((1, 128), lambda i: (0, 0)),
                  pl.BlockSpec((128, 256), lambda i: (0, 0))],
        out_specs=[pl.BlockSpec((_BM, 128), lambda i: (i, 0)),
                   pl.BlockSpec((_BM, 128), lambda i: (i, 0))],
        out_shape=[jax.ShapeDtypeStruct((_NPAD, 128), jnp.float32),
                   jax.ShapeDtypeStruct((_NPAD, 128), jnp.float32)],
    )(accp, denp, brow, wext)


def _final_body(a_ref, d_ref, b_ref, o_ref):
    recip = 1.0 / (d_ref[0][:, 0:1] + _EPS)
    alo, ahi = _halves(a_ref)
    o_ref[:, :_HD] = alo * recip + b_ref[:, :_HD]
    o_ref[:, _HD:] = ahi * recip + b_ref[:, _HD:]


def _final(accp, denp, brow):
    return pl.pallas_call(
        _final_body,
        grid=(_N // _BM,),
        in_specs=[pl.BlockSpec((2, _NB, _BM, 16), lambda i: (0, 0, i, 0)),
                  pl.BlockSpec((2, _BM, 16), lambda i: (0, i, 0)),
                  pl.BlockSpec((1, 128), lambda i: (0, 0))],
        out_specs=pl.BlockSpec((_BM, 128), lambda i: (i, 0)),
        out_shape=jax.ShapeDtypeStruct((_N, 128), jnp.float32),
    )(accp, denp, brow)


def _edge_body(ei_h, pab_h, mx_h, h_h,
               accp, denp,
               pab_v, mx_v, srcb, dstb, idxb, exb, exf,
               rowb, hb0, hb1, hb2, hb3, sem,
               acc_s, den_s):
    c = lax.axis_index("c")
    s = lax.axis_index("s")
    hb = [hb0, hb1, hb2, hb3]

    pltpu.sync_copy(pab_h, pab_v)
    pltpu.sync_copy(mx_h, mx_v)

    mxv = mx_v[...]
    lane = lax.iota(jnp.int32, 16)
    col0 = jnp.zeros((16,), jnp.int32)
    coff = c * _HD
    zero16 = jnp.zeros((16,), jnp.float32)
    cidx = [coff + j * 16 + lane for j in range(_NB)]

    def _fill_idx(base):
        for j in range(_C // 16):
            idxb[pl.ds(j * 16, 16)] = lane + (base + j * 16)

    def _zrow(i, carry):
        exb[i, :] = zero16
        return carry

    lax.fori_loop(0, _C, _zrow, 0)
    for r in range(_SEG // _C):
        for j in range(_NB):
            _fill_idx(j * _NPAD + s * _SEG + r * _C)
            pltpu.sync_copy(exb, acc_s.at[idxb])
        _fill_idx(s * _SEG + r * _C)
        pltpu.sync_copy(exb, den_s.at[idxb])
    plsc.subcore_barrier()

    def _process():
        gcp = pltpu.async_copy(h_h.at[srcb], rowb, sem)
        for j in range(_C // 16):
            sv = srcb[pl.ds(j * 16, 16)]
            dv = dstb[pl.ds(j * 16, 16)]
            ps = plsc.load_gather(pab_v, [sv])
            pd = plsc.load_gather(pab_v, [dv])
            _, a1 = plsc.unpack(plsc.bitcast(ps, jnp.bfloat16),
                                format=plsc.PackFormat.INTERLEAVED)
            a2, _ = plsc.unpack(plsc.bitcast(pd, jnp.bfloat16),
                                format=plsc.PackFormat.INTERLEAVED)
            t = a1 + a2
            e = jnp.maximum(t, 0.2 * t)
            u = mxv + a2
            m = jnp.maximum(u, 0.2 * u)
            ex = jnp.exp(e - m)
            plsc.store_scatter(exb, [lane + j * 16, col0], ex)
            exf[pl.ds(j * 16, 16)] = ex
        gcp.wait()

        def _scale(g2, carry2):
            exv = exf[pl.ds(g2 * 16, 16)]
            for el in range(16):
                i = g2 * 16 + el
                al = lax.broadcast(exv[el], (16,))
                ridx = lax.broadcast(i, (16,))
                for j in range(_NB):
                    v = plsc.load_gather(rowb, [ridx, cidx[j]])
                    hb[j][i, :] = v * al
            return carry2

        lax.fori_loop(0, _C // 16, _scale, 0)
        for j in range(_NB):
            if j:
                for g2 in range(_C // 16):
                    idxb[pl.ds(g2 * 16, 16)] = (
                        dstb[pl.ds(g2 * 16, 16)] + j * _NPAD)
                pltpu.sync_copy(hb[j], acc_s.at[idxb], add=True)
            else:
                pltpu.sync_copy(hb[j], acc_s.at[dstb], add=True)
        pltpu.sync_copy(exb, den_s.at[dstb], add=True)

    def _real_chunk(g, carry):
        base = s * _EPT + g * _C
        pltpu.sync_copy(ei_h.at[0, pl.ds(base, _C)], srcb)
        pltpu.sync_copy(ei_h.at[1, pl.ds(base, _C)], dstb)
        _process()
        return carry

    def _synth_chunk(g, carry):
        base = s * _EPT + g * _C
        for j in range(_C // 16):
            sl = (base - _E + j * 16) + lane
            ok = sl < _N
            srcb[pl.ds(j * 16, 16)] = jnp.where(ok, sl, 0)
            dstb[pl.ds(j * 16, 16)] = jnp.where(ok, sl, _NPAD - 1)
        _process()
        return carry

    nreal = jnp.maximum(
        jnp.minimum((_E - s * _EPT) // _C, _NCH), 0)
    lax.fori_loop(0, nreal, _real_chunk, 0)
    lax.fori_loop(nreal, _NCH, _synth_chunk, 0)
    plsc.subcore_barrier()

    for r in range(_SEG // _C):
        for j in range(_NB):
            _fill_idx(j * _NPAD + s * _SEG + r * _C)
            pltpu.sync_copy(acc_s.at[idxb], hb[j])
            pltpu.sync_copy(
                hb[j], accp.at[c, j, pl.ds(s * _SEG + r * _C, _C), :])
        _fill_idx(s * _SEG + r * _C)
        pltpu.sync_copy(den_s.at[idxb], exb)
        pltpu.sync_copy(exb, denp.at[c, pl.ds(s * _SEG + r * _C, _C), :])


_edge = functools.partial(
    pl.kernel,
    mesh=plsc.VectorSubcoreMesh(core_axis_name="c", subcore_axis_name="s"),
    compiler_params=pltpu.CompilerParams(needs_layout_passes=False),
    out_type=[jax.ShapeDtypeStruct((2, _NB, _NPAD, 16), jnp.float32),
              jax.ShapeDtypeStruct((2, _NPAD, 16), jnp.float32)],
    scratch_types=[
        pltpu.VMEM((_NPAD,), jnp.int32),
        pltpu.VMEM((16,), jnp.float32),
        pltpu.VMEM((_C,), jnp.int32),
        pltpu.VMEM((_C,), jnp.int32),
        pltpu.VMEM((_C,), jnp.int32),
        pltpu.VMEM((_C, 16), jnp.float32),
        pltpu.VMEM((_C,), jnp.float32),
        pltpu.VMEM((_C, 128), jnp.float32),
        pltpu.VMEM((_C, 16), jnp.float32),
        pltpu.VMEM((_C, 16), jnp.float32),
        pltpu.VMEM((_C, 16), jnp.float32),
        pltpu.VMEM((_C, 16), jnp.float32),
        pltpu.SemaphoreType.DMA,
        pltpu.VMEM_SHARED((_NB * _NPAD, 16), jnp.float32),
        pltpu.VMEM_SHARED((_NPAD, 16), jnp.float32),
    ],
)(_edge_body)


def _pack(sca):
    au = jax.lax.bitcast_convert_type(
        sca[:, 0].astype(jnp.bfloat16), jnp.uint16).astype(jnp.uint32)
    du = jax.lax.bitcast_convert_type(
        sca[:, 1].astype(jnp.bfloat16), jnp.uint16).astype(jnp.uint32)
    return jax.lax.bitcast_convert_type(au | (du << 16), jnp.int32)


def _mx(sca):
    a = sca[:, 0]
    m = jnp.max(a) + 0.01 * jnp.max(jnp.abs(a)) + 0.01
    return jnp.full((16,), m, jnp.float32)


def kernel(x, edge_index, W1, a_src1, a_dst1, b1, W2, a_src2, a_dst2, b2):
    zcols = jnp.zeros((128, 126), jnp.float32)
    wext1 = jnp.concatenate([W1, W1 @ a_src1.T, W1 @ a_dst1.T, zcols], axis=1)
    wext2 = jnp.concatenate([W2, W2 @ a_src2.T, W2 @ a_dst2.T, zcols], axis=1)
    b1r = b1.reshape(1, 128)
    b2r = b2.reshape(1, 128)

    h1, sca1 = _proj(x, wext1)
    accp1, denp1 = _edge(edge_index, _pack(sca1), _mx(sca1), h1)

    h2, sca2 = _proj2(accp1, denp1, b1r, wext2)
    accp2, denp2 = _edge(edge_index, _pack(sca2), _mx(sca2), h2)

    return _final(accp2, denp2, b2r)
